# baseline (speedup 1.0000x reference)
import functools

import jax
import jax.numpy as jnp
from jax import lax
from jax.experimental import pallas as pl
from jax.experimental.pallas import tpu as pltpu
from jax.experimental.pallas import tpu_sc as plsc

_N = 10000
_E = 320000
_FEAT = 128
_HID = 128
_OUT = 64
_G = 16

_NC = 2
_NS = 16
_NW = _NC * _NS
_EPW = _E // _NW
_K = 80
_CPW = _EPW // _K
_ROFF = 624
_RSZ = 640


@functools.partial(
    pl.kernel,
    out_type=jax.ShapeDtypeStruct((_NC, _N, _HID), jnp.float32),
    mesh=plsc.VectorSubcoreMesh(core_axis_name="c", subcore_axis_name="s"),
    scratch_types=[
        pltpu.VMEM((_EPW,), jnp.int32),
        pltpu.VMEM((_CPW, _K), jnp.int32),
        pltpu.VMEM((_K, _HID), jnp.float32),
        pltpu.VMEM((_K, _HID), jnp.float32),
        pltpu.VMEM_SHARED((_N, _HID), jnp.float32),
        pltpu.SemaphoreType.DMA,
        pltpu.SemaphoreType.DMA,
        pltpu.SemaphoreType.DMA,
        pltpu.SemaphoreType.DMA,
    ],
)
def _sc_edge_agg(h_hbm, src_hbm, dst_hbm, out_hbm, src_v, dst_v, r0, r1,
                 acc, sg0, sg1, ss0, ss1):
    c = lax.axis_index("c")
    s = lax.axis_index("s")
    wid = s * _NC + c
    row0 = pl.multiple_of(s * _ROFF, 8)
    pltpu.sync_copy(src_hbm.at[pl.ds(pl.multiple_of(wid * _EPW, 8), _EPW)],
                    src_v)
    pltpu.sync_copy(dst_hbm.at[wid], dst_v)

    def _src(i):
        return src_v.at[pl.ds(i * _K, _K)]

    def _g_start(i, buf, sem):
        pltpu.async_copy(h_hbm.at[_src(i)], buf, sem)

    def _g_wait(i, buf, sem):
        pltpu.make_async_copy(h_hbm.at[_src(i)], buf, sem).wait()

    def _s_start(i, buf, sem):
        pltpu.async_copy(buf, acc.at[dst_v.at[i]], sem, add=True)

    def _s_wait(i, buf, sem):
        pltpu.make_async_copy(buf, acc.at[dst_v.at[i]], sem).wait()

    _g_start(0, r0, sg0)
    _g_start(1, r1, sg1)
    pltpu.sync_copy(h_hbm.at[pl.ds(row0, _RSZ)], acc.at[pl.ds(row0, _RSZ)])
    plsc.subcore_barrier()

    def body(i, carry):
        c0 = 2 * i
        c1 = c0 + 1
        _g_wait(c0, r0, sg0)
        _s_start(c0, r0, ss0)
        _g_wait(c1, r1, sg1)
        _s_start(c1, r1, ss1)
        _s_wait(c0, r0, ss0)
        _g_start(c0 + 2, r0, sg0)
        _s_wait(c1, r1, ss1)
        _g_start(c1 + 2, r1, sg1)
        return carry

    lax.fori_loop(0, (_CPW - 3) // 2, body, 0)
    _g_wait(_CPW - 3, r0, sg0)
    _s_start(_CPW - 3, r0, ss0)
    _g_wait(_CPW - 2, r1, sg1)
    _s_start(_CPW - 2, r1, ss1)
    _s_wait(_CPW - 3, r0, ss0)
    _g_start(_CPW - 1, r0, sg0)
    _s_wait(_CPW - 2, r1, ss1)
    _g_wait(_CPW - 1, r0, sg0)
    pltpu.sync_copy(r0, acc.at[dst_v.at[_CPW - 1]], add=True)
    plsc.subcore_barrier()
    pltpu.sync_copy(acc.at[pl.ds(row0, _RSZ)],
                    out_hbm.at[c, pl.ds(row0, _RSZ)])


def _mlp_bn(z, w1, b1, w2, b2, g, bt):
    a = jnp.maximum(jnp.dot(z, w1, preferred_element_type=jnp.float32) + b1,
                    0.0)
    z2 = jnp.dot(a, w2, preferred_element_type=jnp.float32) + b2
    mu = jnp.mean(z2, axis=0, keepdims=True)
    zc = z2 - mu
    var = jnp.mean(zc * zc, axis=0, keepdims=True)
    zn = zc * lax.rsqrt(var + 1e-5) * g + bt
    return jnp.maximum(zn, 0.0)


def _tc_layer_body(p_ref, h_ref, w1_ref, b1_ref, w2_ref, b2_ref, g_ref,
                   bt_ref, o_ref):
    z = p_ref[0] + p_ref[1] - h_ref[...]
    o_ref[...] = _mlp_bn(z, w1_ref[...], b1_ref[...], w2_ref[...],
                         b2_ref[...], g_ref[...], bt_ref[...])


_tc_layer = pl.pallas_call(
    _tc_layer_body,
    out_shape=jax.ShapeDtypeStruct((_N, _HID), jnp.float32),
)


def _tc_final_body(p_ref, h_ref, w1_ref, b1_ref, w2_ref, b2_ref, g_ref,
                   bt_ref, batch_ref, wf1_ref, bf1_ref, wf2_ref, bf2_ref,
                   o_ref):
    z = p_ref[0] + p_ref[1] - h_ref[...]
    hl = _mlp_bn(z, w1_ref[...], b1_ref[...], w2_ref[...], b2_ref[...],
                 g_ref[...], bt_ref[...])
    onehot_t = (lax.broadcasted_iota(jnp.int32, (_G, _N), 0)
                == batch_ref[...]).astype(jnp.float32)
    pooled = jnp.dot(onehot_t, hl, preferred_element_type=jnp.float32)
    f1 = jnp.maximum(
        jnp.dot(pooled, wf1_ref[...], preferred_element_type=jnp.float32)
        + bf1_ref[...], 0.0)
    o_ref[...] = (jnp.dot(f1, wf2_ref[...],
                          preferred_element_type=jnp.float32)
                  + bf2_ref[...])


_tc_final = pl.pallas_call(
    _tc_final_body,
    out_shape=jax.ShapeDtypeStruct((_G, _OUT), jnp.float32),
)


def kernel(x, edge_index, batch, params):
    ei = jnp.asarray(edge_index, jnp.int32)
    src = ei[0]
    dst = ei[1].reshape(_NW, _CPW, _K)
    batch2d = jnp.asarray(batch, jnp.int32).reshape(1, _N)
    h = x
    n_layers = len(params["layers"])
    for i, lp in enumerate(params["layers"]):
        p = _sc_edge_agg(h, src, dst)
        w1 = lp["W1"]
        b1 = lp["b1"].reshape(1, _HID)
        w2 = lp["W2"]
        b2 = lp["b2"].reshape(1, _HID)
        g = lp["gamma"].reshape(1, _HID)
        bt = lp["beta"].reshape(1, _HID)
        if i < n_layers - 1:
            h = _tc_layer(p, h, w1, b1, w2, b2, g, bt)
        else:
            fc = params["fc"]
            out = _tc_final(p, h, w1, b1, w2, b2, g, bt, batch2d,
                            fc["Wf1"], fc["bf1"].reshape(1, _HID),
                            fc["Wf2"], fc["bf2"].reshape(1, _OUT))
    return out


# speedup vs baseline: 1.2555x; 1.2555x over previous
import functools

import jax
import jax.numpy as jnp
from jax import lax
from jax.experimental import pallas as pl
from jax.experimental.pallas import tpu as pltpu
from jax.experimental.pallas import tpu_sc as plsc

_N = 10000
_E = 320000
_FEAT = 128
_HID = 128
_OUT = 64
_G = 16

_NC = 2
_NS = 16
_NW = _NC * _NS
_EPW = _E // _NW
_K = 80
_CPW = _EPW // _K
_ROFF = 624
_RSZ = 640


@functools.partial(
    pl.kernel,
    out_type=jax.ShapeDtypeStruct((_NC, _N, _HID), jnp.float32),
    mesh=plsc.VectorSubcoreMesh(core_axis_name="c", subcore_axis_name="s"),
    scratch_types=[
        pltpu.VMEM((_EPW,), jnp.int32),
        pltpu.VMEM((_CPW, _K), jnp.int32),
        pltpu.VMEM((_K, _HID), jnp.float32),
        pltpu.VMEM((_K, _HID), jnp.float32),
        pltpu.VMEM_SHARED((_N, _HID), jnp.float32),
        pltpu.SemaphoreType.DMA,
        pltpu.SemaphoreType.DMA,
    ],
)
def _sc_edge_agg(h_hbm, src_hbm, dst_hbm, out_hbm, src_v, dst_v, r0, r1,
                 acc, sg0, sg1):
    c = lax.axis_index("c")
    s = lax.axis_index("s")
    wid = s * _NC + c
    row0 = pl.multiple_of(s * _ROFF, 8)
    pltpu.sync_copy(src_hbm.at[pl.ds(pl.multiple_of(wid * _EPW, 8), _EPW)],
                    src_v)
    pltpu.sync_copy(dst_hbm.at[wid], dst_v)

    def _src(i):
        return src_v.at[pl.ds(i * _K, _K)]

    def _g_start(i, buf, sem):
        pltpu.async_copy(h_hbm.at[_src(i)], buf, sem)

    def _g_wait(i, buf, sem):
        pltpu.make_async_copy(h_hbm.at[_src(i)], buf, sem).wait()

    _g_start(0, r0, sg0)
    pltpu.sync_copy(h_hbm.at[pl.ds(row0, _RSZ)], acc.at[pl.ds(row0, _RSZ)])
    plsc.subcore_barrier()

    def body(i, carry):
        i0 = 2 * i
        _g_start(i0 + 1, r1, sg1)
        _g_wait(i0, r0, sg0)
        pltpu.sync_copy(r0, acc.at[dst_v.at[i0]], add=True)
        _g_start(i0 + 2, r0, sg0)
        _g_wait(i0 + 1, r1, sg1)
        pltpu.sync_copy(r1, acc.at[dst_v.at[i0 + 1]], add=True)
        return carry

    lax.fori_loop(0, (_CPW - 3) // 2, body, 0)
    _g_start(_CPW - 2, r1, sg1)
    _g_wait(_CPW - 3, r0, sg0)
    pltpu.sync_copy(r0, acc.at[dst_v.at[_CPW - 3]], add=True)
    _g_start(_CPW - 1, r0, sg0)
    _g_wait(_CPW - 2, r1, sg1)
    pltpu.sync_copy(r1, acc.at[dst_v.at[_CPW - 2]], add=True)
    _g_wait(_CPW - 1, r0, sg0)
    pltpu.sync_copy(r0, acc.at[dst_v.at[_CPW - 1]], add=True)
    plsc.subcore_barrier()
    pltpu.sync_copy(acc.at[pl.ds(row0, _RSZ)],
                    out_hbm.at[c, pl.ds(row0, _RSZ)])


def _mlp_bn(z, w1, b1, w2, b2, g, bt):
    a = jnp.maximum(jnp.dot(z, w1, preferred_element_type=jnp.float32) + b1,
                    0.0)
    z2 = jnp.dot(a, w2, preferred_element_type=jnp.float32) + b2
    mu = jnp.mean(z2, axis=0, keepdims=True)
    zc = z2 - mu
    var = jnp.mean(zc * zc, axis=0, keepdims=True)
    zn = zc * lax.rsqrt(var + 1e-5) * g + bt
    return jnp.maximum(zn, 0.0)


def _tc_layer_body(p_ref, h_ref, w1_ref, b1_ref, w2_ref, b2_ref, g_ref,
                   bt_ref, o_ref):
    z = p_ref[0] + p_ref[1] - h_ref[...]
    o_ref[...] = _mlp_bn(z, w1_ref[...], b1_ref[...], w2_ref[...],
                         b2_ref[...], g_ref[...], bt_ref[...])


_tc_layer = pl.pallas_call(
    _tc_layer_body,
    out_shape=jax.ShapeDtypeStruct((_N, _HID), jnp.float32),
)


def _tc_final_body(p_ref, h_ref, w1_ref, b1_ref, w2_ref, b2_ref, g_ref,
                   bt_ref, batch_ref, wf1_ref, bf1_ref, wf2_ref, bf2_ref,
                   o_ref):
    z = p_ref[0] + p_ref[1] - h_ref[...]
    hl = _mlp_bn(z, w1_ref[...], b1_ref[...], w2_ref[...], b2_ref[...],
                 g_ref[...], bt_ref[...])
    onehot_t = (lax.broadcasted_iota(jnp.int32, (_G, _N), 0)
                == batch_ref[...]).astype(jnp.float32)
    pooled = jnp.dot(onehot_t, hl, preferred_element_type=jnp.float32)
    f1 = jnp.maximum(
        jnp.dot(pooled, wf1_ref[...], preferred_element_type=jnp.float32)
        + bf1_ref[...], 0.0)
    o_ref[...] = (jnp.dot(f1, wf2_ref[...],
                          preferred_element_type=jnp.float32)
                  + bf2_ref[...])


_tc_final = pl.pallas_call(
    _tc_final_body,
    out_shape=jax.ShapeDtypeStruct((_G, _OUT), jnp.float32),
)


def kernel(x, edge_index, batch, params):
    ei = jnp.asarray(edge_index, jnp.int32)
    src = ei[0]
    dst = ei[1].reshape(_NW, _CPW, _K)
    batch2d = jnp.asarray(batch, jnp.int32).reshape(1, _N)
    h = x
    n_layers = len(params["layers"])
    for i, lp in enumerate(params["layers"]):
        p = _sc_edge_agg(h, src, dst)
        w1 = lp["W1"]
        b1 = lp["b1"].reshape(1, _HID)
        w2 = lp["W2"]
        b2 = lp["b2"].reshape(1, _HID)
        g = lp["gamma"].reshape(1, _HID)
        bt = lp["beta"].reshape(1, _HID)
        if i < n_layers - 1:
            h = _tc_layer(p, h, w1, b1, w2, b2, g, bt)
        else:
            fc = params["fc"]
            out = _tc_final(p, h, w1, b1, w2, b2, g, bt, batch2d,
                            fc["Wf1"], fc["bf1"].reshape(1, _HID),
                            fc["Wf2"], fc["bf2"].reshape(1, _OUT))
    return out


# speedup vs baseline: 1.5148x; 1.2065x over previous
import functools

import jax
import jax.numpy as jnp
from jax import lax
from jax.experimental import pallas as pl
from jax.experimental.pallas import tpu as pltpu
from jax.experimental.pallas import tpu_sc as plsc

_N = 10000
_E = 320000
_FEAT = 128
_HID = 128
_OUT = 64
_G = 16

_NC = 2
_NS = 16
_NW = _NC * _NS
_EPW = _E // _NW
_K = 80
_CPW = _EPW // _K
_ROFF = 624
_RSZ = 640


@functools.partial(
    pl.kernel,
    out_type=jax.ShapeDtypeStruct((_NC, _N, _HID), jnp.float32),
    mesh=plsc.VectorSubcoreMesh(core_axis_name="c", subcore_axis_name="s"),
    scratch_types=[
        pltpu.VMEM((_EPW,), jnp.int32),
        pltpu.VMEM((_EPW,), jnp.int32),
        pltpu.VMEM((_K,), jnp.int32),
        pltpu.VMEM((_K, _HID), jnp.float32),
        pltpu.VMEM((_K, _HID), jnp.float32),
        pltpu.VMEM((_K, _HID), jnp.float32),
        pltpu.VMEM_SHARED((_N, _HID), jnp.float32),
        pltpu.SemaphoreType.DMA,
        pltpu.SemaphoreType.DMA,
        pltpu.SemaphoreType.DMA,
    ],
)
def _sc_edge_agg(h_hbm, src_hbm, dst_hbm, out_hbm, src_v, dst_v,
                 cur, r0, r1, r2, acc, sg0, sg1, sg2):
    c = lax.axis_index("c")
    s = lax.axis_index("s")
    wid = s * _NC + c
    row0 = pl.multiple_of(s * _ROFF, 8)
    base = pl.multiple_of(wid * _EPW, 8)
    pltpu.sync_copy(src_hbm.at[pl.ds(base, _EPW)], src_v)
    pltpu.sync_copy(dst_hbm.at[pl.ds(base, _EPW)], dst_v)

    def _src(i):
        return src_v.at[pl.ds(i * _K, _K)]

    def _g_start(i, buf, sem):
        pltpu.async_copy(h_hbm.at[_src(i)], buf, sem)

    def _g_wait(i, buf, sem):
        pltpu.make_async_copy(h_hbm.at[_src(i)], buf, sem).wait()

    def _stage_dst(i):
        off = pl.multiple_of(i * _K, 16)
        for j in range(_K // 16):
            cur[pl.ds(16 * j, 16)] = dst_v[pl.ds(off + 16 * j, 16)]

    def _chunk(i, buf, sem, start_i, start_buf, start_sem):
        if start_i is not None:
            _g_start(start_i, start_buf, start_sem)
        _stage_dst(i)
        _g_wait(i, buf, sem)
        pltpu.sync_copy(buf, acc.at[cur], add=True)

    _g_start(0, r0, sg0)
    _g_start(1, r1, sg1)
    pltpu.sync_copy(h_hbm.at[pl.ds(row0, _RSZ)], acc.at[pl.ds(row0, _RSZ)])
    plsc.subcore_barrier()

    def body(k, carry):
        i0 = 3 * k
        _chunk(i0, r0, sg0, i0 + 2, r2, sg2)
        _chunk(i0 + 1, r1, sg1, i0 + 3, r0, sg0)
        _chunk(i0 + 2, r2, sg2, i0 + 4, r1, sg1)
        return carry

    lax.fori_loop(0, (_CPW - 2) // 3, body, 0)
    _chunk(_CPW - 2, r0, sg0, None, None, None)
    _chunk(_CPW - 1, r1, sg1, None, None, None)
    plsc.subcore_barrier()
    pltpu.sync_copy(acc.at[pl.ds(row0, _RSZ)],
                    out_hbm.at[c, pl.ds(row0, _RSZ)])


def _mlp_bn(z, w1, b1, w2, b2, g, bt):
    a = jnp.maximum(jnp.dot(z, w1, preferred_element_type=jnp.float32) + b1,
                    0.0)
    z2 = jnp.dot(a, w2, preferred_element_type=jnp.float32) + b2
    mu = jnp.mean(z2, axis=0, keepdims=True)
    zc = z2 - mu
    var = jnp.mean(zc * zc, axis=0, keepdims=True)
    zn = zc * lax.rsqrt(var + 1e-5) * g + bt
    return jnp.maximum(zn, 0.0)


def _tc_layer_body(p_ref, h_ref, w1_ref, b1_ref, w2_ref, b2_ref, g_ref,
                   bt_ref, o_ref):
    z = p_ref[0] + p_ref[1] - h_ref[...]
    o_ref[...] = _mlp_bn(z, w1_ref[...], b1_ref[...], w2_ref[...],
                         b2_ref[...], g_ref[...], bt_ref[...])


_tc_layer = pl.pallas_call(
    _tc_layer_body,
    out_shape=jax.ShapeDtypeStruct((_N, _HID), jnp.float32),
)


def _tc_final_body(p_ref, h_ref, w1_ref, b1_ref, w2_ref, b2_ref, g_ref,
                   bt_ref, batch_ref, wf1_ref, bf1_ref, wf2_ref, bf2_ref,
                   o_ref):
    z = p_ref[0] + p_ref[1] - h_ref[...]
    hl = _mlp_bn(z, w1_ref[...], b1_ref[...], w2_ref[...], b2_ref[...],
                 g_ref[...], bt_ref[...])
    onehot_t = (lax.broadcasted_iota(jnp.int32, (_G, _N), 0)
                == batch_ref[...]).astype(jnp.float32)
    pooled = jnp.dot(onehot_t, hl, preferred_element_type=jnp.float32)
    f1 = jnp.maximum(
        jnp.dot(pooled, wf1_ref[...], preferred_element_type=jnp.float32)
        + bf1_ref[...], 0.0)
    o_ref[...] = (jnp.dot(f1, wf2_ref[...],
                          preferred_element_type=jnp.float32)
                  + bf2_ref[...])


_tc_final = pl.pallas_call(
    _tc_final_body,
    out_shape=jax.ShapeDtypeStruct((_G, _OUT), jnp.float32),
)


def kernel(x, edge_index, batch, params):
    ei = jnp.asarray(edge_index, jnp.int32)
    src = ei[0]
    dst = ei[1]
    batch2d = jnp.asarray(batch, jnp.int32).reshape(1, _N)
    h = x
    n_layers = len(params["layers"])
    for i, lp in enumerate(params["layers"]):
        p = _sc_edge_agg(h, src, dst)
        w1 = lp["W1"]
        b1 = lp["b1"].reshape(1, _HID)
        w2 = lp["W2"]
        b2 = lp["b2"].reshape(1, _HID)
        g = lp["gamma"].reshape(1, _HID)
        bt = lp["beta"].reshape(1, _HID)
        if i < n_layers - 1:
            h = _tc_layer(p, h, w1, b1, w2, b2, g, bt)
        else:
            fc = params["fc"]
            out = _tc_final(p, h, w1, b1, w2, b2, g, bt, batch2d,
                            fc["Wf1"], fc["bf1"].reshape(1, _HID),
                            fc["Wf2"], fc["bf2"].reshape(1, _OUT))
    return out


# speedup vs baseline: 1.5437x; 1.0191x over previous
import functools

import jax
import jax.numpy as jnp
from jax import lax
from jax.experimental import pallas as pl
from jax.experimental.pallas import tpu as pltpu
from jax.experimental.pallas import tpu_sc as plsc

_N = 10000
_E = 320000
_FEAT = 128
_HID = 128
_OUT = 64
_G = 16

_NC = 2
_NS = 16
_NW = _NC * _NS
_EPW = _E // _NW
_K = 80
_CPW = _EPW // _K
_ROFF = 624
_RSZ = 640


@functools.partial(
    pl.kernel,
    out_type=jax.ShapeDtypeStruct((_NC, _N, _HID), jnp.float32),
    mesh=plsc.VectorSubcoreMesh(core_axis_name="c", subcore_axis_name="s"),
    scratch_types=[
        pltpu.VMEM((_EPW,), jnp.int32),
        pltpu.VMEM((_EPW,), jnp.int32),
        pltpu.VMEM((_K,), jnp.int32),
        pltpu.VMEM((_K, _HID), jnp.float32),
        pltpu.VMEM((_K, _HID), jnp.float32),
        pltpu.VMEM((_K, _HID), jnp.float32),
        pltpu.VMEM_SHARED((_N, _HID), jnp.float32),
        pltpu.SemaphoreType.DMA,
        pltpu.SemaphoreType.DMA,
        pltpu.SemaphoreType.DMA,
        pltpu.SemaphoreType.DMA,
        pltpu.SemaphoreType.DMA,
        pltpu.SemaphoreType.DMA,
    ],
)
def _sc_edge_agg(h_hbm, src_hbm, dst_hbm, out_hbm, src_v, dst_v,
                 cur, r0, r1, r2, acc, sg0, sg1, sg2, sp, sp2, sp3):
    c = lax.axis_index("c")
    s = lax.axis_index("s")
    wid = s * _NC + c
    row0 = pl.multiple_of(s * _ROFF, 8)
    base = pl.multiple_of(wid * _EPW, 8)
    pltpu.async_copy(src_hbm.at[pl.ds(base, _EPW)], src_v, sp)
    pltpu.async_copy(dst_hbm.at[pl.ds(base, _EPW)], dst_v, sp2)
    pltpu.async_copy(h_hbm.at[pl.ds(row0, _RSZ)], acc.at[pl.ds(row0, _RSZ)],
                     sp3)

    def _src(i):
        return src_v.at[pl.ds(i * _K, _K)]

    def _g_start(i, buf, sem):
        pltpu.async_copy(h_hbm.at[_src(i)], buf, sem)

    def _g_wait(i, buf, sem):
        pltpu.make_async_copy(h_hbm.at[_src(i)], buf, sem).wait()

    def _stage_dst(i):
        off = pl.multiple_of(i * _K, 16)
        for j in range(_K // 16):
            cur[pl.ds(16 * j, 16)] = dst_v[pl.ds(off + 16 * j, 16)]

    def _chunk(i, buf, sem, start_i, start_buf, start_sem):
        if start_i is not None:
            _g_start(start_i, start_buf, start_sem)
        _stage_dst(i)
        _g_wait(i, buf, sem)
        pltpu.sync_copy(buf, acc.at[cur], add=True)

    pltpu.make_async_copy(src_hbm.at[pl.ds(base, _EPW)], src_v, sp).wait()
    _g_start(0, r0, sg0)
    _g_start(1, r1, sg1)
    pltpu.make_async_copy(dst_hbm.at[pl.ds(base, _EPW)], dst_v, sp2).wait()
    pltpu.make_async_copy(h_hbm.at[pl.ds(row0, _RSZ)],
                          acc.at[pl.ds(row0, _RSZ)], sp3).wait()
    plsc.subcore_barrier()

    def body(k, carry):
        i0 = 3 * k
        _chunk(i0, r0, sg0, i0 + 2, r2, sg2)
        _chunk(i0 + 1, r1, sg1, i0 + 3, r0, sg0)
        _chunk(i0 + 2, r2, sg2, i0 + 4, r1, sg1)
        return carry

    lax.fori_loop(0, (_CPW - 2) // 3, body, 0)
    _chunk(_CPW - 2, r0, sg0, None, None, None)
    _chunk(_CPW - 1, r1, sg1, None, None, None)
    plsc.subcore_barrier()
    pltpu.sync_copy(acc.at[pl.ds(row0, _RSZ)],
                    out_hbm.at[c, pl.ds(row0, _RSZ)])


def _mlp_bn(z, w1, b1, w2, b2, g, bt):
    a = jnp.maximum(jnp.dot(z, w1, preferred_element_type=jnp.float32) + b1,
                    0.0)
    z2 = jnp.dot(a, w2, preferred_element_type=jnp.float32) + b2
    mu = jnp.mean(z2, axis=0, keepdims=True)
    zc = z2 - mu
    var = jnp.mean(zc * zc, axis=0, keepdims=True)
    zn = zc * lax.rsqrt(var + 1e-5) * g + bt
    return jnp.maximum(zn, 0.0)


def _tc_layer_body(p_ref, h_ref, w1_ref, b1_ref, w2_ref, b2_ref, g_ref,
                   bt_ref, o_ref):
    z = p_ref[0] + p_ref[1] - h_ref[...]
    o_ref[...] = _mlp_bn(z, w1_ref[...], b1_ref[...], w2_ref[...],
                         b2_ref[...], g_ref[...], bt_ref[...])


_tc_layer = pl.pallas_call(
    _tc_layer_body,
    out_shape=jax.ShapeDtypeStruct((_N, _HID), jnp.float32),
)


def _tc_final_body(p_ref, h_ref, w1_ref, b1_ref, w2_ref, b2_ref, g_ref,
                   bt_ref, batch_ref, wf1_ref, bf1_ref, wf2_ref, bf2_ref,
                   o_ref):
    z = p_ref[0] + p_ref[1] - h_ref[...]
    hl = _mlp_bn(z, w1_ref[...], b1_ref[...], w2_ref[...], b2_ref[...],
                 g_ref[...], bt_ref[...])
    onehot_t = (lax.broadcasted_iota(jnp.int32, (_G, _N), 0)
                == batch_ref[...]).astype(jnp.float32)
    pooled = jnp.dot(onehot_t, hl, preferred_element_type=jnp.float32)
    f1 = jnp.maximum(
        jnp.dot(pooled, wf1_ref[...], preferred_element_type=jnp.float32)
        + bf1_ref[...], 0.0)
    o_ref[...] = (jnp.dot(f1, wf2_ref[...],
                          preferred_element_type=jnp.float32)
                  + bf2_ref[...])


_tc_final = pl.pallas_call(
    _tc_final_body,
    out_shape=jax.ShapeDtypeStruct((_G, _OUT), jnp.float32),
)


def kernel(x, edge_index, batch, params):
    ei = jnp.asarray(edge_index, jnp.int32)
    src = ei[0]
    dst = ei[1]
    batch2d = jnp.asarray(batch, jnp.int32).reshape(1, _N)
    h = x
    n_layers = len(params["layers"])
    for i, lp in enumerate(params["layers"]):
        p = _sc_edge_agg(h, src, dst)
        w1 = lp["W1"]
        b1 = lp["b1"].reshape(1, _HID)
        w2 = lp["W2"]
        b2 = lp["b2"].reshape(1, _HID)
        g = lp["gamma"].reshape(1, _HID)
        bt = lp["beta"].reshape(1, _HID)
        if i < n_layers - 1:
            h = _tc_layer(p, h, w1, b1, w2, b2, g, bt)
        else:
            fc = params["fc"]
            out = _tc_final(p, h, w1, b1, w2, b2, g, bt, batch2d,
                            fc["Wf1"], fc["bf1"].reshape(1, _HID),
                            fc["Wf2"], fc["bf2"].reshape(1, _OUT))
    return out
